# baseline (speedup 1.0000x reference)
import functools

import jax
import jax.numpy as jnp
from jax import lax
from jax.experimental import pallas as pl
from jax.experimental.pallas import tpu as pltpu
from jax.experimental.pallas import tpu_sc as plsc

N = 10000
E = 320000
D = 128
H = 256

NC = 2
NS = 16
NW = NC * NS
CHUNK = 72
PAIR_CHUNKS = 280
K0 = 120
K1 = PAIR_CHUNKS - K0
TOTAL_CHUNKS = NS * PAIR_CHUNKS
E_PER_PAIR = CHUNK * PAIR_CHUNKS
E_PAD = E_PER_PAIR * NS
ROWS_MAIN = 624
ROWS_LAST = N - 15 * ROWS_MAIN
X_PAD_ROWS = 8
EPS = 1e-5


def _sc_partial_segment_sum(x_p, src_w, dst_w, zeros):
    mesh = plsc.VectorSubcoreMesh(core_axis_name="c", subcore_axis_name="s")

    @functools.partial(
        pl.kernel,
        out_type=jax.ShapeDtypeStruct((NC * N, D), jnp.float32),
        mesh=mesh,
        scratch_types=[
            pltpu.VMEM((K1 * CHUNK,), jnp.int32),
            pltpu.VMEM((K1, CHUNK), jnp.int32),
            pltpu.VMEM((2, CHUNK, D), jnp.float32),
            pltpu.VMEM_SHARED((N, D), jnp.float32),
            pltpu.SemaphoreType.DMA,
            pltpu.SemaphoreType.DMA,
        ],
    )
    def sc_kernel(x_hbm, src_hbm, dst_hbm, z_hbm, out_hbm,
                  src_v, dst_v, rows_v, acc, sem0, sem1):
        c = lax.axis_index("c")
        s = lax.axis_index("s")
        r0 = s * ROWS_MAIN
        base_chunk = s * PAIR_CHUNKS + c * K0
        nchunks = jnp.where(c == 0, K0, K1)
        @pl.when(s < NS - 1)
        def _():
            pltpu.sync_copy(z_hbm.at[pl.ds(r0, ROWS_MAIN)],
                            acc.at[pl.ds(r0, ROWS_MAIN)])

        @pl.when(s == NS - 1)
        def _():
            pltpu.sync_copy(z_hbm.at[pl.ds(15 * ROWS_MAIN, ROWS_LAST)],
                            acc.at[pl.ds(15 * ROWS_MAIN, ROWS_LAST)])

        @pl.when(c == 0)
        def _():
            pltpu.sync_copy(src_hbm.at[pl.ds(base_chunk * CHUNK, K0 * CHUNK)],
                            src_v.at[pl.ds(0, K0 * CHUNK)])
            pltpu.sync_copy(dst_hbm.at[pl.ds(base_chunk, K0)],
                            dst_v.at[pl.ds(0, K0)])

        @pl.when(c == 1)
        def _():
            pltpu.sync_copy(src_hbm.at[pl.ds(base_chunk * CHUNK, K1 * CHUNK)],
                            src_v)
            pltpu.sync_copy(dst_hbm.at[pl.ds(base_chunk, K1)], dst_v)
        plsc.subcore_barrier()

        sems = (sem0, sem1)

        def start_gather(j, b):
            pltpu.async_copy(x_hbm.at[src_v.at[pl.ds(j * CHUNK, CHUNK)]],
                             rows_v.at[b], sems[b])

        for b in range(2):
            start_gather(b, b)

        def body(i, carry):
            for b in range(2):
                j = 2 * i + b
                pltpu.make_async_copy(
                    x_hbm.at[src_v.at[pl.ds(j * CHUNK, CHUNK)]],
                    rows_v.at[b], sems[b]).wait()
                pltpu.sync_copy(rows_v.at[b], acc.at[dst_v.at[j]], add=True)
                nxt = j + 2

                @pl.when(nxt < nchunks)
                def _():
                    start_gather(nxt, b)
            return carry

        lax.fori_loop(0, nchunks // 2, body, 0)
        plsc.subcore_barrier()

        @pl.when(s < NS - 1)
        def _():
            pltpu.sync_copy(acc.at[pl.ds(r0, ROWS_MAIN)],
                            out_hbm.at[pl.ds(c * N + r0, ROWS_MAIN)])

        @pl.when(s == NS - 1)
        def _():
            pltpu.sync_copy(acc.at[pl.ds(15 * ROWS_MAIN, ROWS_LAST)],
                            out_hbm.at[pl.ds(c * N + 15 * ROWS_MAIN, ROWS_LAST)])

    return sc_kernel(x_p, src_w, dst_w, zeros)


def _tc_dense(x, parts, W_root, W_nbr, b_base, gamma1, beta1,
              W1, b1, W2, b2, gamma2, beta2):
    def body(x_ref, parts_ref, wr_ref, wn_ref, bb_ref, g1_ref, be1_ref,
             w1_ref, b1_ref, w2_ref, b2_ref, g2_ref, be2_ref, o_ref):
        xv = x_ref[...]
        agg = parts_ref[0:N, :] + parts_ref[N:2 * N, :]
        h = jnp.dot(xv, wr_ref[...], preferred_element_type=jnp.float32)
        h = h + jnp.dot(agg, wn_ref[...], preferred_element_type=jnp.float32)
        h = h + bb_ref[...]
        mu = jnp.mean(h, axis=0, keepdims=True)
        hc = h - mu
        var = jnp.mean(hc * hc, axis=0, keepdims=True)
        h = hc * lax.rsqrt(var + EPS) * g1_ref[...] + be1_ref[...]
        h = jnp.maximum(h, 0.0) + xv
        t = jnp.maximum(
            jnp.dot(h, w1_ref[...], preferred_element_type=jnp.float32)
            + b1_ref[...], 0.0)
        y = (jnp.dot(t, w2_ref[...], preferred_element_type=jnp.float32)
             + b2_ref[...] + h)
        mu2 = jnp.mean(y, axis=0, keepdims=True)
        yc = y - mu2
        var2 = jnp.mean(yc * yc, axis=0, keepdims=True)
        o_ref[...] = yc * lax.rsqrt(var2 + EPS) * g2_ref[...] + be2_ref[...]

    return pl.pallas_call(
        body,
        out_shape=jax.ShapeDtypeStruct((N, D), jnp.float32),
    )(x, parts, W_root, W_nbr,
      b_base.reshape(1, D), gamma1.reshape(1, D), beta1.reshape(1, D),
      W1, b1.reshape(1, H), W2, b2.reshape(1, D),
      gamma2.reshape(1, D), beta2.reshape(1, D))


def kernel(x, edge_index, W_root, W_nbr, b_base, gamma1, beta1,
           W1, b1, W2, b2, gamma2, beta2):
    src = edge_index[0]
    dst = edge_index[1]
    pad = E_PAD - E
    src_w = jnp.concatenate([src, jnp.full((pad,), N, jnp.int32)])
    dst_w = jnp.concatenate(
        [dst, jnp.zeros((pad,), jnp.int32)]).reshape(TOTAL_CHUNKS, CHUNK)
    x_p = jnp.concatenate([x, jnp.zeros((X_PAD_ROWS, D), jnp.float32)])
    zeros = jnp.zeros((N, D), jnp.float32)
    parts = _sc_partial_segment_sum(x_p, src_w, dst_w, zeros)
    return _tc_dense(x, parts, W_root, W_nbr, b_base, gamma1, beta1,
                     W1, b1, W2, b2, gamma2, beta2)
